# Initial kernel scaffold; baseline (speedup 1.0000x reference)
import jax
import jax.numpy as jnp
from jax.experimental import pallas as pl


def kernel(x, edge_index, batch, W1, b1, W2, b2, W3, b3, Wc1, bc1, Wc2, bc2):
    raise NotImplementedError("write your pallas kernel here")


# speedup vs baseline: 13.8907x; 13.8907x over previous
import functools

import jax
import jax.numpy as jnp
from jax import lax
from jax.experimental import pallas as pl
from jax.experimental.pallas import tpu as pltpu
from jax.experimental.pallas import tpu_sc as plsc

N = 10000
F = 128
H = 64
G = 64
E = 320000

NC = 2
NS = 16
NW = NC * NS

CH = 128
EPAD = 327680
CPW = EPAD // (NW * CH)
NPAD = 10240
RPT = NPAD // NS
DW = 8

BN = 1024
NBLK = NPAD // BN

_mesh = plsc.VectorSubcoreMesh(
    core_axis_name="c", subcore_axis_name="s", num_cores=NC, num_subcores=NS)


@functools.partial(
    pl.kernel,
    out_type=jax.ShapeDtypeStruct((NC, NPAD, DW), jnp.float32),
    mesh=_mesh,
    compiler_params=pltpu.CompilerParams(use_tc_tiling_on_sc=False),
    scratch_types=[
        pltpu.VMEM((CPW, CH), jnp.int32),
        pltpu.VMEM((CH, DW), jnp.float32),
        pltpu.VMEM((CH, DW), jnp.float32),
        pltpu.VMEM_SHARED((NPAD, DW), jnp.float32),
    ],
)
def _sc_degree(dst_hbm, ones_hbm, zeros_hbm, out_hbm, dstb, onesb, zerob, acc):
    c = lax.axis_index("c")
    s = lax.axis_index("s")
    w = s * NC + c
    pltpu.sync_copy(dst_hbm.at[pl.ds(w * CPW, CPW)], dstb)
    pltpu.sync_copy(ones_hbm, onesb)
    pltpu.sync_copy(zeros_hbm, zerob)
    for k in range(RPT // CH):
        pltpu.sync_copy(zerob, acc.at[pl.ds(s * RPT + k * CH, CH)])
    plsc.subcore_barrier()

    @pl.loop(0, CPW)
    def _(j):
        pltpu.sync_copy(onesb, acc.at[dstb.at[j]], add=True)

    plsc.subcore_barrier()
    for k in range(RPT // CH):
        r0 = s * RPT + k * CH
        pltpu.sync_copy(acc.at[pl.ds(r0, CH)], out_hbm.at[c, pl.ds(r0, CH)])


@functools.partial(
    pl.kernel,
    out_type=jax.ShapeDtypeStruct((NC, NPAD, H), jnp.float32),
    mesh=_mesh,
    compiler_params=pltpu.CompilerParams(use_tc_tiling_on_sc=False),
    scratch_types=[
        pltpu.VMEM((CPW, CH), jnp.int32),
        pltpu.VMEM((CPW, CH), jnp.int32),
        pltpu.VMEM((2, CH, H), jnp.float32),
        pltpu.VMEM((CH, H), jnp.float32),
        pltpu.VMEM_SHARED((NPAD, H), jnp.float32),
        pltpu.SemaphoreType.DMA,
        pltpu.SemaphoreType.DMA,
    ],
)
def _sc_scatter(hs_hbm, src_hbm, dst_hbm, zeros_hbm, out_hbm,
                srcb, dstb, rows, zerob, acc, gsem, ssem):
    c = lax.axis_index("c")
    s = lax.axis_index("s")
    w = s * NC + c
    pltpu.sync_copy(src_hbm.at[pl.ds(w * CPW, CPW)], srcb)
    pltpu.sync_copy(dst_hbm.at[pl.ds(w * CPW, CPW)], dstb)
    pltpu.sync_copy(zeros_hbm, zerob)
    for k in range(RPT // CH):
        pltpu.sync_copy(zerob, acc.at[pl.ds(s * RPT + k * CH, CH)])
    plsc.subcore_barrier()

    @pl.loop(0, CPW, step=2)
    def _(j):
        g0 = pltpu.async_copy(hs_hbm.at[srcb.at[j]], rows.at[0], gsem)
        g1 = pltpu.async_copy(hs_hbm.at[srcb.at[j + 1]], rows.at[1], gsem)
        g0.wait()
        s0 = pltpu.async_copy(rows.at[0], acc.at[dstb.at[j]], ssem, add=True)
        g1.wait()
        s1 = pltpu.async_copy(rows.at[1], acc.at[dstb.at[j + 1]], ssem, add=True)
        s0.wait()
        s1.wait()

    plsc.subcore_barrier()
    for k in range(RPT // CH):
        r0 = s * RPT + k * CH
        pltpu.sync_copy(acc.at[pl.ds(r0, CH)], out_hbm.at[c, pl.ds(r0, CH)])


def _tc_first_body(x_ref, w_ref, degp_ref, hs_ref, d_ref):
    deg = degp_ref[0, :, 0:1] + degp_ref[1, :, 0:1] + 1.0
    dinv = lax.rsqrt(deg)
    h = jnp.dot(x_ref[...], w_ref[...], preferred_element_type=jnp.float32)
    hs_ref[...] = h * dinv
    d_ref[...] = jnp.broadcast_to(dinv, (BN, H))


_tc_first = pl.pallas_call(
    _tc_first_body,
    grid=(NBLK,),
    in_specs=[
        pl.BlockSpec((BN, F), lambda i: (i, 0)),
        pl.BlockSpec((F, H), lambda i: (0, 0)),
        pl.BlockSpec((NC, BN, DW), lambda i: (0, i, 0)),
    ],
    out_specs=[
        pl.BlockSpec((BN, H), lambda i: (i, 0)),
        pl.BlockSpec((BN, H), lambda i: (i, 0)),
    ],
    out_shape=[
        jax.ShapeDtypeStruct((NPAD, H), jnp.float32),
        jax.ShapeDtypeStruct((NPAD, H), jnp.float32),
    ],
)


def _tc_mid_body(accp_ref, hs_ref, d_ref, b_ref, w_ref, out_ref):
    a = accp_ref[0] + accp_ref[1] + hs_ref[...]
    t = jnp.maximum(d_ref[...] * a + b_ref[...], 0.0)
    h = jnp.dot(t, w_ref[...], preferred_element_type=jnp.float32)
    out_ref[...] = h * d_ref[...]


_tc_mid = pl.pallas_call(
    _tc_mid_body,
    grid=(NBLK,),
    in_specs=[
        pl.BlockSpec((NC, BN, H), lambda i: (0, i, 0)),
        pl.BlockSpec((BN, H), lambda i: (i, 0)),
        pl.BlockSpec((BN, H), lambda i: (i, 0)),
        pl.BlockSpec((1, H), lambda i: (0, 0)),
        pl.BlockSpec((H, H), lambda i: (0, 0)),
    ],
    out_specs=pl.BlockSpec((BN, H), lambda i: (i, 0)),
    out_shape=jax.ShapeDtypeStruct((NPAD, H), jnp.float32),
)


def _tc_final_body(accp_ref, hs_ref, d_ref, b_ref, batch_ref,
                   wc1_ref, bc1_ref, wc2_ref, bc2_ref, out_ref,
                   sums_ref, cnt_ref):
    i = pl.program_id(0)

    @pl.when(i == 0)
    def _():
        sums_ref[...] = jnp.zeros_like(sums_ref)
        cnt_ref[...] = jnp.zeros_like(cnt_ref)

    t3 = d_ref[...] * (accp_ref[0] + accp_ref[1] + hs_ref[...]) + b_ref[...]
    bt = batch_ref[0, 0, :].reshape(BN, 1)
    oh = (bt == lax.broadcasted_iota(jnp.int32, (1, G), 1)).astype(jnp.float32)
    sums_ref[...] += lax.dot_general(
        oh, t3, (((0,), (0,)), ((), ())), preferred_element_type=jnp.float32)
    cnt_ref[...] += lax.dot_general(
        oh, jnp.ones((BN, 1), jnp.float32), (((0,), (0,)), ((), ())),
        preferred_element_type=jnp.float32)

    @pl.when(i == NBLK - 1)
    def _():
        pooled = sums_ref[...] / jnp.maximum(cnt_ref[...], 1.0)
        z = jnp.dot(pooled, wc1_ref[...], preferred_element_type=jnp.float32)
        z = jnp.maximum(z + bc1_ref[...], 0.0)
        z2 = jnp.dot(z, wc2_ref[...], preferred_element_type=jnp.float32)
        z2 = z2 + bc2_ref[...]
        out_ref[...] = 1.0 / (1.0 + jnp.exp(-z2))


_tc_final = pl.pallas_call(
    _tc_final_body,
    grid=(NBLK,),
    in_specs=[
        pl.BlockSpec((NC, BN, H), lambda i: (0, i, 0)),
        pl.BlockSpec((BN, H), lambda i: (i, 0)),
        pl.BlockSpec((BN, H), lambda i: (i, 0)),
        pl.BlockSpec((1, H), lambda i: (0, 0)),
        pl.BlockSpec((1, 1, BN), lambda i: (i, 0, 0)),
        pl.BlockSpec((H, 32), lambda i: (0, 0)),
        pl.BlockSpec((1, 32), lambda i: (0, 0)),
        pl.BlockSpec((32, 1), lambda i: (0, 0)),
        pl.BlockSpec((1, 1), lambda i: (0, 0)),
    ],
    out_specs=pl.BlockSpec((G, 1), lambda i: (0, 0)),
    out_shape=jax.ShapeDtypeStruct((G, 1), jnp.float32),
    scratch_shapes=[
        pltpu.VMEM((G, H), jnp.float32),
        pltpu.VMEM((G, 1), jnp.float32),
    ],
)


def kernel(x, edge_index, batch, W1, b1, W2, b2, W3, b3, Wc1, bc1, Wc2, bc2):
    src, dst = edge_index[0], edge_index[1]
    pad_e = jnp.full((EPAD - E,), N, jnp.int32)
    src_p = jnp.concatenate([src, pad_e]).reshape(NW * CPW, CH)
    dst_p = jnp.concatenate([dst, pad_e]).reshape(NW * CPW, CH)
    x_p = jnp.pad(x, ((0, NPAD - N), (0, 0)))
    batch_p = jnp.concatenate(
        [batch, jnp.full((NPAD - N,), G, jnp.int32)]).reshape(NBLK, 1, BN)

    ones_dw = jnp.ones((CH, DW), jnp.float32)
    zeros_dw = jnp.zeros((CH, DW), jnp.float32)
    zeros_h = jnp.zeros((CH, H), jnp.float32)

    degp = _sc_degree(dst_p, ones_dw, zeros_dw)
    hs1, d = _tc_first(x_p, W1, degp)
    p1 = _sc_scatter(hs1, src_p, dst_p, zeros_h)
    hs2 = _tc_mid(p1, hs1, d, b1.reshape(1, H), W2)
    p2 = _sc_scatter(hs2, src_p, dst_p, zeros_h)
    hs3 = _tc_mid(p2, hs2, d, b2.reshape(1, H), W3)
    p3 = _sc_scatter(hs3, src_p, dst_p, zeros_h)
    out = _tc_final(p3, hs3, d, b3.reshape(1, H), batch_p,
                    Wc1, bc1.reshape(1, 32), Wc2, bc2.reshape(1, 1))
    return out
